# baseline (speedup 1.0000x reference)
import functools

import jax
import jax.numpy as jnp
from jax import lax
from jax.experimental import pallas as pl
from jax.experimental.pallas import tpu as pltpu
from jax.experimental.pallas import tpu_sc as plsc

CORPUS = 1000000
B = 16384
D = 64
NNEG = 10
NC = 2
NS = 16
NW = NC * NS
EPW = B // NW
CB = 64
NCH = EPW // CB
NIDXROW = 128
NROWS = CB * NNEG
NJ = NROWS // NIDXROW
VW = 2 * D


def _sc_body(vrow_u, par_u, vrow_i, par_i, vrow_n, par_n, tab_u, tab_i,
             pos_out, neg_out,
             idxu_v, paru_v, idxi_v, pari_v, idxn_v, parn_v,
             u_rows, i_rows, n_rows, pos_v, neg_v,
             sem):
  cid = lax.axis_index("c")
  sid = lax.axis_index("s")
  wid = sid * NC + cid

  iota = lax.iota(jnp.int32, 16)

  def chunk_body(c, carry):
    base_e = (wid * NCH + c) * CB
    pltpu.sync_copy(vrow_u.at[pl.ds(base_e, CB)], idxu_v)
    pltpu.sync_copy(par_u.at[pl.ds(base_e, CB)], paru_v)
    pltpu.sync_copy(vrow_i.at[pl.ds(base_e, CB)], idxi_v)
    pltpu.sync_copy(par_i.at[pl.ds(base_e, CB)], pari_v)
    pltpu.sync_copy(vrow_n.at[pl.ds(base_e * NNEG, NROWS)], idxn_v)
    pltpu.sync_copy(par_n.at[pl.ds(base_e * NNEG, NROWS)], parn_v)

    cps = [
        pltpu.async_copy(tab_u.at[idxu_v], u_rows, sem),
        pltpu.async_copy(tab_i.at[idxi_v], i_rows, sem),
    ]
    for j in range(NJ):
      cps.append(
          pltpu.async_copy(
              tab_i.at[idxn_v.at[pl.ds(j * NIDXROW, NIDXROW)]],
              n_rows.at[pl.ds(j * NIDXROW, NIDXROW)], sem))
    for cp in cps:
      cp.wait()

    for g in range(CB // 16):
      rowu = iota + g * 16
      rowns = [rowu * NNEG + n for n in range(NNEG)]
      paru = paru_v[pl.ds(g * 16, 16)]
      pari = pari_v[pl.ds(g * 16, 16)]
      parns = [plsc.load_gather(parn_v, [rowns[n]]) for n in range(NNEG)]

      def d_body(d, acc):
        accp, accn = acc
        u_d = plsc.load_gather(u_rows, [rowu, paru + d])
        i_d = plsc.load_gather(i_rows, [rowu, pari + d])
        accp = accp + u_d * i_d
        accn = tuple(
            accn[n] + plsc.load_gather(n_rows, [rowns[n], parns[n] + d]) * u_d
            for n in range(NNEG))
        return accp, accn

      z = jnp.zeros((16,), jnp.float32)
      accp, accn = lax.fori_loop(0, D, d_body, (z, (z,) * NNEG))
      pos_v[pl.ds(g * 16, 16)] = accp
      for n in range(NNEG):
        neg_v[n, pl.ds(g * 16, 16)] = accn[n]

    pltpu.sync_copy(pos_v, pos_out.at[pl.ds(base_e, CB)])
    pltpu.sync_copy(neg_v, neg_out.at[wid, c])
    return carry

  lax.fori_loop(0, NCH, chunk_body, 0)


_sc_scores = functools.partial(
    pl.kernel,
    mesh=plsc.VectorSubcoreMesh(
        core_axis_name="c", subcore_axis_name="s",
        num_cores=NC, num_subcores=NS),
    out_type=[
        jax.ShapeDtypeStruct((B,), jnp.float32),
        jax.ShapeDtypeStruct((NW, NCH, NNEG, CB), jnp.float32),
    ],
    scratch_types=[
        pltpu.VMEM((CB,), jnp.int32),
        pltpu.VMEM((CB,), jnp.int32),
        pltpu.VMEM((CB,), jnp.int32),
        pltpu.VMEM((CB,), jnp.int32),
        pltpu.VMEM((NROWS,), jnp.int32),
        pltpu.VMEM((NROWS,), jnp.int32),
        pltpu.VMEM((CB, VW), jnp.float32),
        pltpu.VMEM((CB, VW), jnp.float32),
        pltpu.VMEM((NROWS, VW), jnp.float32),
        pltpu.VMEM((CB,), jnp.float32),
        pltpu.VMEM((NNEG, CB), jnp.float32),
        pltpu.SemaphoreType.DMA,
    ],
    compiler_params=pltpu.CompilerParams(needs_layout_passes=False),
)(_sc_body)


def _tc_finish(pos_ref, neg_ref, out_ref):
  s = jnp.clip(pos_ref[...], -10.0, 10.0)
  t = jnp.clip(neg_ref[...], -10.0, 10.0)
  total = jnp.sum(jax.nn.softplus(-s)) + jnp.sum(jax.nn.softplus(t))
  out_ref[0, 0] = total * (1.0 / B)


def kernel(pos_input, pos_item, neg_item, input_emb, item_emb):
  pos_idx = pos_input.astype(jnp.int32)
  item_idx = pos_item.astype(jnp.int32)
  neg_idx = neg_item.astype(jnp.int32).reshape(B * NNEG)

  vrow_u = pos_idx >> 1
  par_u = (pos_idx & 1) << 6
  vrow_i = item_idx >> 1
  par_i = (item_idx & 1) << 6
  vrow_n = neg_idx >> 1
  par_n = (neg_idx & 1) << 6

  tab_u = input_emb.reshape(CORPUS // 2, VW)
  tab_i = item_emb.reshape(CORPUS // 2, VW)

  pos_sc, neg_sc = _sc_scores(vrow_u, par_u, vrow_i, par_i, vrow_n, par_n,
                              tab_u, tab_i)

  out = pl.pallas_call(
      _tc_finish,
      out_shape=jax.ShapeDtypeStruct((1, 1), jnp.float32),
      in_specs=[
          pl.BlockSpec(memory_space=pltpu.VMEM),
          pl.BlockSpec(memory_space=pltpu.VMEM),
      ],
      out_specs=pl.BlockSpec(memory_space=pltpu.SMEM),
  )(pos_sc.reshape(128, 128), neg_sc.reshape(B * NNEG // 128, 128))
  return out[0, 0]


# speedup vs baseline: 1.6507x; 1.6507x over previous
import functools

import jax
import jax.numpy as jnp
from jax import lax
from jax.experimental import pallas as pl
from jax.experimental.pallas import tpu as pltpu
from jax.experimental.pallas import tpu_sc as plsc

CORPUS = 1000000
B = 16384
D = 64
NNEG = 10
NC = 2
NS = 16
NW = NC * NS
EPW = B // NW
CB = 64
NCH = EPW // CB
NIDXROW = 128
NROWS = CB * NNEG
NJ = NROWS // NIDXROW
VW = 2 * D


def _sc_body(vrow_u, par_u, vrow_i, par_i, vrow_n, par_n, tab_u, tab_i,
             pos_out, neg_out,
             idxu_v, paru_v, idxi_v, pari_v, idxn_v, parn_v,
             u_rows, i_rows, n_rows, pos_v, neg_v,
             sem):
  cid = lax.axis_index("c")
  sid = lax.axis_index("s")
  wid = sid * NC + cid

  iota = lax.iota(jnp.int32, 16)

  def chunk_body(c, carry):
    base_e = (wid * NCH + c) * CB
    pltpu.sync_copy(vrow_u.at[pl.ds(base_e, CB)], idxu_v)
    pltpu.sync_copy(par_u.at[pl.ds(base_e, CB)], paru_v)
    pltpu.sync_copy(vrow_i.at[pl.ds(base_e, CB)], idxi_v)
    pltpu.sync_copy(par_i.at[pl.ds(base_e, CB)], pari_v)
    pltpu.sync_copy(vrow_n.at[pl.ds(base_e * NNEG, NROWS)], idxn_v)
    pltpu.sync_copy(par_n.at[pl.ds(base_e * NNEG, NROWS)], parn_v)

    cps = [
        pltpu.async_copy(tab_u.at[idxu_v], u_rows, sem),
        pltpu.async_copy(tab_i.at[idxi_v], i_rows, sem),
    ]
    for j in range(NJ):
      cps.append(
          pltpu.async_copy(
              tab_i.at[idxn_v.at[pl.ds(j * NIDXROW, NIDXROW)]],
              n_rows.at[pl.ds(j * NIDXROW, NIDXROW)], sem))
    for cp in cps:
      cp.wait()

    for g in range(CB // 16):
      rowu = iota + g * 16
      rowns = [rowu * NNEG + n for n in range(NNEG)]
      paru = paru_v[pl.ds(g * 16, 16)]
      pari = pari_v[pl.ds(g * 16, 16)]
      parns = [plsc.load_gather(parn_v, [rowns[n]]) for n in range(NNEG)]

      def d_body(d, acc):
        accp, accn = acc
        u_d = plsc.load_gather(u_rows, [rowu, paru + d])
        i_d = plsc.load_gather(i_rows, [rowu, pari + d])
        accp = accp + u_d * i_d
        accn = tuple(
            accn[n] + plsc.load_gather(n_rows, [rowns[n], parns[n] + d]) * u_d
            for n in range(NNEG))
        return accp, accn

      z = jnp.zeros((16,), jnp.float32)
      accp, accn = lax.fori_loop(0, D, d_body, (z, (z,) * NNEG))
      pos_v[pl.ds(g * 16, 16)] = accp
      for n in range(NNEG):
        neg_v[n, pl.ds(g * 16, 16)] = accn[n]

    pltpu.sync_copy(pos_v, pos_out.at[pl.ds(base_e, CB)])
    pltpu.sync_copy(neg_v, neg_out.at[wid, c])
    return carry

  lax.fori_loop(0, NCH, chunk_body, 0)


_sc_scores = functools.partial(
    pl.kernel,
    mesh=plsc.VectorSubcoreMesh(
        core_axis_name="c", subcore_axis_name="s",
        num_cores=NC, num_subcores=NS),
    out_type=[
        jax.ShapeDtypeStruct((B,), jnp.float32),
        jax.ShapeDtypeStruct((NW, NCH, NNEG, CB), jnp.float32),
    ],
    scratch_types=[
        pltpu.VMEM((CB,), jnp.int32),
        pltpu.VMEM((CB,), jnp.int32),
        pltpu.VMEM((CB,), jnp.int32),
        pltpu.VMEM((CB,), jnp.int32),
        pltpu.VMEM((NROWS,), jnp.int32),
        pltpu.VMEM((NROWS,), jnp.int32),
        pltpu.VMEM((CB, VW), jnp.float32),
        pltpu.VMEM((CB, VW), jnp.float32),
        pltpu.VMEM((NROWS, VW), jnp.float32),
        pltpu.VMEM((CB,), jnp.float32),
        pltpu.VMEM((NNEG, CB), jnp.float32),
        pltpu.SemaphoreType.DMA,
    ],
    compiler_params=pltpu.CompilerParams(needs_layout_passes=False),
)(_sc_body)


TBLK = 8192
TG = (CORPUS + TBLK - 1) // TBLK
VROWS = TG * (TBLK // 2)


def _transpose_body(embt_ref, out_ref):
  x = embt_ref[...]
  out_ref[:, 0:D] = jnp.transpose(x[:, : TBLK // 2], (1, 0))
  out_ref[:, D:VW] = jnp.transpose(x[:, TBLK // 2 :], (1, 0))


_repack = pl.pallas_call(
    _transpose_body,
    grid=(TG,),
    in_specs=[pl.BlockSpec((D, TBLK), lambda g: (0, g))],
    out_specs=pl.BlockSpec((TBLK // 2, VW), lambda g: (g, 0)),
    out_shape=jax.ShapeDtypeStruct((VROWS, VW), jnp.float32),
)


def _tc_finish(pos_ref, neg_ref, out_ref):
  s = jnp.clip(pos_ref[...], -10.0, 10.0)
  t = jnp.clip(neg_ref[...], -10.0, 10.0)
  total = jnp.sum(jax.nn.softplus(-s)) + jnp.sum(jax.nn.softplus(t))
  out_ref[0, 0] = total * (1.0 / B)


def kernel(pos_input, pos_item, neg_item, input_emb, item_emb):
  pos_idx = pos_input.astype(jnp.int32)
  item_idx = pos_item.astype(jnp.int32)
  neg_idx = neg_item.astype(jnp.int32).reshape(B * NNEG)

  def _vrow(r):
    return (r >> 13) * (TBLK // 2) + (r & (TBLK // 2 - 1))

  def _par(r):
    return (r & (TBLK // 2)) >> 6

  vrow_u = _vrow(pos_idx)
  par_u = _par(pos_idx)
  vrow_i = _vrow(item_idx)
  par_i = _par(item_idx)
  vrow_n = _vrow(neg_idx)
  par_n = _par(neg_idx)

  tab_u = _repack(input_emb.T)
  tab_i = _repack(item_emb.T)

  pos_sc, neg_sc = _sc_scores(vrow_u, par_u, vrow_i, par_i, vrow_n, par_n,
                              tab_u, tab_i)

  out = pl.pallas_call(
      _tc_finish,
      out_shape=jax.ShapeDtypeStruct((1, 1), jnp.float32),
      in_specs=[
          pl.BlockSpec(memory_space=pltpu.VMEM),
          pl.BlockSpec(memory_space=pltpu.VMEM),
      ],
      out_specs=pl.BlockSpec(memory_space=pltpu.SMEM),
  )(pos_sc.reshape(128, 128), neg_sc.reshape(B * NNEG // 128, 128))
  return out[0, 0]


# speedup vs baseline: 1.7301x; 1.0481x over previous
import functools

import jax
import jax.numpy as jnp
from jax import lax
from jax.experimental import pallas as pl
from jax.experimental.pallas import tpu as pltpu
from jax.experimental.pallas import tpu_sc as plsc

CORPUS = 1000000
B = 16384
D = 64
NNEG = 10
NC = 2
NS = 16
NW = NC * NS
EPW = B // NW
CB = 32
NCH = EPW // CB
NIDXROW = 128
NROWS = CB * NNEG
NJ = NROWS // NIDXROW
VW = 2 * D
UNROLL = 4


def _sc_body(vrow_u, par_u, vrow_i, par_i, vrow_n, par_n, tab_u, tab_i,
             pos_out, neg_out,
             idxu_v, paru_v, idxi_v, pari_v, idxn_v, parn_v,
             u_rows, i_rows, n_rows, pos_v, neg_v,
             sem0, sem1):
  cid = lax.axis_index("c")
  sid = lax.axis_index("s")
  wid = sid * NC + cid
  sems = (sem0, sem1)

  iota = lax.iota(jnp.int32, 16)

  def stage_idx(b, c):
    base_e = (wid * NCH + c) * CB
    pltpu.sync_copy(vrow_u.at[pl.ds(base_e, CB)], idxu_v.at[pl.ds(b * CB, CB)])
    pltpu.sync_copy(par_u.at[pl.ds(base_e, CB)], paru_v.at[pl.ds(b * CB, CB)])
    pltpu.sync_copy(vrow_i.at[pl.ds(base_e, CB)], idxi_v.at[pl.ds(b * CB, CB)])
    pltpu.sync_copy(par_i.at[pl.ds(base_e, CB)], pari_v.at[pl.ds(b * CB, CB)])
    pltpu.sync_copy(vrow_n.at[pl.ds(base_e * NNEG, NROWS)],
                    idxn_v.at[pl.ds(b * NROWS, NROWS)])
    pltpu.sync_copy(par_n.at[pl.ds(base_e * NNEG, NROWS)],
                    parn_v.at[pl.ds(b * NROWS, NROWS)])

  def row_copies(b):
    cps = [
        pltpu.make_async_copy(tab_u.at[idxu_v.at[pl.ds(b * CB, CB)]],
                              u_rows.at[pl.ds(b * CB, CB)], sems[b]),
        pltpu.make_async_copy(tab_i.at[idxi_v.at[pl.ds(b * CB, CB)]],
                              i_rows.at[pl.ds(b * CB, CB)], sems[b]),
    ]
    for j in range(0, NROWS, NIDXROW):
      w = min(NIDXROW, NROWS - j)
      cps.append(
          pltpu.make_async_copy(
              tab_i.at[idxn_v.at[pl.ds(b * NROWS + j, w)]],
              n_rows.at[pl.ds(b * NROWS + j, w)], sems[b]))
    return cps

  def fire(b):
    for cp in row_copies(b):
      cp.start()

  def drain(b):
    for cp in row_copies(b):
      cp.wait()

  def compute(b, c):
    base_e = (wid * NCH + c) * CB
    for g in range(CB // 16):
      rowu = iota + g * 16 + b * CB
      rowns = [(iota + g * 16) * NNEG + n + b * NROWS for n in range(NNEG)]
      paru = paru_v[pl.ds(b * CB + g * 16, 16)]
      pari = pari_v[pl.ds(b * CB + g * 16, 16)]
      parns = [plsc.load_gather(parn_v, [rowns[n]]) for n in range(NNEG)]

      def d_body(i, acc):
        accp, accn = acc
        d0 = i * UNROLL
        for k in range(UNROLL):
          d = d0 + k
          u_d = plsc.load_gather(u_rows, [rowu, paru + d])
          i_d = plsc.load_gather(i_rows, [rowu, pari + d])
          accp = accp + u_d * i_d
          accn = tuple(
              accn[n]
              + plsc.load_gather(n_rows, [rowns[n], parns[n] + d]) * u_d
              for n in range(NNEG))
        return accp, accn

      z = jnp.zeros((16,), jnp.float32)
      accp, accn = lax.fori_loop(0, D // UNROLL, d_body, (z, (z,) * NNEG))
      pos_v[pl.ds(g * 16, 16)] = accp
      for n in range(NNEG):
        neg_v[n, pl.ds(g * 16, 16)] = accn[n]

    pltpu.sync_copy(pos_v, pos_out.at[pl.ds(base_e, CB)])
    pltpu.sync_copy(neg_v, neg_out.at[wid, c])

  stage_idx(0, 0)
  fire(0)
  stage_idx(1, 1)
  fire(1)

  def pair_body(p, carry):
    for b in range(2):
      c = p * 2 + b
      drain(b)
      compute(b, c)

      @pl.when(c + 2 < NCH)
      def _():
        stage_idx(b, c + 2)
        fire(b)

    return carry

  lax.fori_loop(0, NCH // 2, pair_body, 0)


_sc_scores = functools.partial(
    pl.kernel,
    mesh=plsc.VectorSubcoreMesh(
        core_axis_name="c", subcore_axis_name="s",
        num_cores=NC, num_subcores=NS),
    out_type=[
        jax.ShapeDtypeStruct((B,), jnp.float32),
        jax.ShapeDtypeStruct((NW, NCH, NNEG, CB), jnp.float32),
    ],
    scratch_types=[
        pltpu.VMEM((2 * CB,), jnp.int32),
        pltpu.VMEM((2 * CB,), jnp.int32),
        pltpu.VMEM((2 * CB,), jnp.int32),
        pltpu.VMEM((2 * CB,), jnp.int32),
        pltpu.VMEM((2 * NROWS,), jnp.int32),
        pltpu.VMEM((2 * NROWS,), jnp.int32),
        pltpu.VMEM((2 * CB, VW), jnp.float32),
        pltpu.VMEM((2 * CB, VW), jnp.float32),
        pltpu.VMEM((2 * NROWS, VW), jnp.float32),
        pltpu.VMEM((CB,), jnp.float32),
        pltpu.VMEM((NNEG, CB), jnp.float32),
        pltpu.SemaphoreType.DMA,
        pltpu.SemaphoreType.DMA,
    ],
    compiler_params=pltpu.CompilerParams(needs_layout_passes=False),
)(_sc_body)


TBLK = 8192
TG = (CORPUS + TBLK - 1) // TBLK
VROWS = TG * (TBLK // 2)


def _transpose_body(embt_ref, out_ref):
  x = embt_ref[...]
  out_ref[:, 0:D] = jnp.transpose(x[:, : TBLK // 2], (1, 0))
  out_ref[:, D:VW] = jnp.transpose(x[:, TBLK // 2 :], (1, 0))


_repack = pl.pallas_call(
    _transpose_body,
    grid=(TG,),
    in_specs=[pl.BlockSpec((D, TBLK), lambda g: (0, g))],
    out_specs=pl.BlockSpec((TBLK // 2, VW), lambda g: (g, 0)),
    out_shape=jax.ShapeDtypeStruct((VROWS, VW), jnp.float32),
)


def _tc_finish(pos_ref, neg_ref, out_ref):
  s = jnp.clip(pos_ref[...], -10.0, 10.0)
  t = jnp.clip(neg_ref[...], -10.0, 10.0)
  total = jnp.sum(jax.nn.softplus(-s)) + jnp.sum(jax.nn.softplus(t))
  out_ref[0, 0] = total * (1.0 / B)


def kernel(pos_input, pos_item, neg_item, input_emb, item_emb):
  pos_idx = pos_input.astype(jnp.int32)
  item_idx = pos_item.astype(jnp.int32)
  neg_idx = neg_item.astype(jnp.int32).reshape(B * NNEG)

  def _vrow(r):
    return (r >> 13) * (TBLK // 2) + (r & (TBLK // 2 - 1))

  def _par(r):
    return (r & (TBLK // 2)) >> 6

  vrow_u = _vrow(pos_idx)
  par_u = _par(pos_idx)
  vrow_i = _vrow(item_idx)
  par_i = _par(item_idx)
  vrow_n = _vrow(neg_idx)
  par_n = _par(neg_idx)

  tab_u = _repack(input_emb.T)
  tab_i = _repack(item_emb.T)

  pos_sc, neg_sc = _sc_scores(vrow_u, par_u, vrow_i, par_i, vrow_n, par_n,
                              tab_u, tab_i)

  out = pl.pallas_call(
      _tc_finish,
      out_shape=jax.ShapeDtypeStruct((1, 1), jnp.float32),
      in_specs=[
          pl.BlockSpec(memory_space=pltpu.VMEM),
          pl.BlockSpec(memory_space=pltpu.VMEM),
      ],
      out_specs=pl.BlockSpec(memory_space=pltpu.SMEM),
  )(pos_sc.reshape(128, 128), neg_sc.reshape(B * NNEG // 128, 128))
  return out[0, 0]


# speedup vs baseline: 2.1710x; 1.2548x over previous
import functools

import jax
import jax.numpy as jnp
from jax import lax
from jax.experimental import pallas as pl
from jax.experimental.pallas import tpu as pltpu
from jax.experimental.pallas import tpu_sc as plsc

CORPUS = 1000000
B = 16384
D = 64
NNEG = 10
NC = 2
NS = 16
NW = NC * NS
EPW = B // NW
CB = 32
NCH = EPW // CB
NIDXROW = 128
NROWS = CB * NNEG
NJ = NROWS // NIDXROW
VW = 2 * D
PHASE = 17
UNROLL = 4


def _sc_body(vrow_u, par_u, vrow_i, par_i, vrow_n, par_n, tab_u, tab_i,
             pos_out, neg_out,
             idxu_v, paru_v, idxi_v, pari_v, idxn_v, parn_v,
             u_rows, i_rows, n_rows, pos_v, neg_v,
             sem0, sem1):
  cid = lax.axis_index("c")
  sid = lax.axis_index("s")
  wid = sid * NC + cid
  sems = (sem0, sem1)

  iota = lax.iota(jnp.int32, 16)

  def stage_idx(b, c):
    base_e = (wid * NCH + c) * CB
    pltpu.sync_copy(vrow_u.at[pl.ds(base_e, CB)], idxu_v.at[pl.ds(b * CB, CB)])
    pltpu.sync_copy(par_u.at[pl.ds(base_e, CB)], paru_v.at[pl.ds(b * CB, CB)])
    pltpu.sync_copy(vrow_i.at[pl.ds(base_e, CB)], idxi_v.at[pl.ds(b * CB, CB)])
    pltpu.sync_copy(par_i.at[pl.ds(base_e, CB)], pari_v.at[pl.ds(b * CB, CB)])
    pltpu.sync_copy(vrow_n.at[pl.ds(base_e * NNEG, NROWS)],
                    idxn_v.at[pl.ds(b * NROWS, NROWS)])
    pltpu.sync_copy(par_n.at[pl.ds(base_e * NNEG, NROWS)],
                    parn_v.at[pl.ds(b * NROWS, NROWS)])

  def row_copies(b):
    cps = [
        pltpu.make_async_copy(tab_u.at[idxu_v.at[pl.ds(b * CB, CB)]],
                              u_rows.at[pl.ds(b * CB, CB)], sems[b]),
        pltpu.make_async_copy(tab_i.at[idxi_v.at[pl.ds(b * CB, CB)]],
                              i_rows.at[pl.ds(b * CB, CB)], sems[b]),
    ]
    for j in range(0, NROWS, NIDXROW):
      w = min(NIDXROW, NROWS - j)
      cps.append(
          pltpu.make_async_copy(
              tab_i.at[idxn_v.at[pl.ds(b * NROWS + j, w)]],
              n_rows.at[pl.ds(b * NROWS + j, w)], sems[b]))
    return cps

  def fire(b):
    for cp in row_copies(b):
      cp.start()

  def drain(b):
    for cp in row_copies(b):
      cp.wait()

  def compute(b, c):
    base_e = (wid * NCH + c) * CB
    for g in range(CB // 16):
      rowu = iota + g * 16 + b * CB
      rowns = [(iota + g * 16) * NNEG + n + b * NROWS for n in range(NNEG)]
      paru = paru_v[pl.ds(b * CB + g * 16, 16)]
      pari = pari_v[pl.ds(b * CB + g * 16, 16)]
      parns = [plsc.load_gather(parn_v, [rowns[n]]) for n in range(NNEG)]

      iotap = iota * PHASE

      def d_body(i, acc):
        accp, accn = acc
        d0 = i * UNROLL
        for k in range(UNROLL):
          dv = (iotap + (d0 + k)) & (D - 1)
          u_d = plsc.load_gather(u_rows, [rowu, paru + dv])
          i_d = plsc.load_gather(i_rows, [rowu, pari + dv])
          accp = accp + u_d * i_d
          accn = tuple(
              accn[n]
              + plsc.load_gather(n_rows, [rowns[n], parns[n] + dv]) * u_d
              for n in range(NNEG))
        return accp, accn

      z = jnp.zeros((16,), jnp.float32)
      accp, accn = lax.fori_loop(0, D // UNROLL, d_body, (z, (z,) * NNEG))
      pos_v[pl.ds(g * 16, 16)] = accp
      for n in range(NNEG):
        neg_v[n, pl.ds(g * 16, 16)] = accn[n]

    pltpu.sync_copy(pos_v, pos_out.at[pl.ds(base_e, CB)])
    pltpu.sync_copy(neg_v, neg_out.at[wid, c])

  stage_idx(0, 0)
  fire(0)
  stage_idx(1, 1)
  fire(1)

  def pair_body(p, carry):
    for b in range(2):
      c = p * 2 + b
      drain(b)
      compute(b, c)

      @pl.when(c + 2 < NCH)
      def _():
        stage_idx(b, c + 2)
        fire(b)

    return carry

  lax.fori_loop(0, NCH // 2, pair_body, 0)


_sc_scores = functools.partial(
    pl.kernel,
    mesh=plsc.VectorSubcoreMesh(
        core_axis_name="c", subcore_axis_name="s",
        num_cores=NC, num_subcores=NS),
    out_type=[
        jax.ShapeDtypeStruct((B,), jnp.float32),
        jax.ShapeDtypeStruct((NW, NCH, NNEG, CB), jnp.float32),
    ],
    scratch_types=[
        pltpu.VMEM((2 * CB,), jnp.int32),
        pltpu.VMEM((2 * CB,), jnp.int32),
        pltpu.VMEM((2 * CB,), jnp.int32),
        pltpu.VMEM((2 * CB,), jnp.int32),
        pltpu.VMEM((2 * NROWS,), jnp.int32),
        pltpu.VMEM((2 * NROWS,), jnp.int32),
        pltpu.VMEM((2 * CB, VW), jnp.float32),
        pltpu.VMEM((2 * CB, VW), jnp.float32),
        pltpu.VMEM((2 * NROWS, VW), jnp.float32),
        pltpu.VMEM((CB,), jnp.float32),
        pltpu.VMEM((NNEG, CB), jnp.float32),
        pltpu.SemaphoreType.DMA,
        pltpu.SemaphoreType.DMA,
    ],
    compiler_params=pltpu.CompilerParams(needs_layout_passes=False),
)(_sc_body)


TBLK = 8192
TG = (CORPUS + TBLK - 1) // TBLK
VROWS = TG * (TBLK // 2)


def _transpose_body(embt_ref, out_ref):
  x = embt_ref[...]
  out_ref[:, 0:D] = jnp.transpose(x[:, : TBLK // 2], (1, 0))
  out_ref[:, D:VW] = jnp.transpose(x[:, TBLK // 2 :], (1, 0))


_repack = pl.pallas_call(
    _transpose_body,
    grid=(TG,),
    in_specs=[pl.BlockSpec((D, TBLK), lambda g: (0, g))],
    out_specs=pl.BlockSpec((TBLK // 2, VW), lambda g: (g, 0)),
    out_shape=jax.ShapeDtypeStruct((VROWS, VW), jnp.float32),
)


def _tc_finish(pos_ref, neg_ref, out_ref):
  s = jnp.clip(pos_ref[...], -10.0, 10.0)
  t = jnp.clip(neg_ref[...], -10.0, 10.0)
  total = jnp.sum(jax.nn.softplus(-s)) + jnp.sum(jax.nn.softplus(t))
  out_ref[0, 0] = total * (1.0 / B)


def kernel(pos_input, pos_item, neg_item, input_emb, item_emb):
  pos_idx = pos_input.astype(jnp.int32)
  item_idx = pos_item.astype(jnp.int32)
  neg_idx = neg_item.astype(jnp.int32).reshape(B * NNEG)

  def _vrow(r):
    return (r >> 13) * (TBLK // 2) + (r & (TBLK // 2 - 1))

  def _par(r):
    return (r & (TBLK // 2)) >> 6

  vrow_u = _vrow(pos_idx)
  par_u = _par(pos_idx)
  vrow_i = _vrow(item_idx)
  par_i = _par(item_idx)
  vrow_n = _vrow(neg_idx)
  par_n = _par(neg_idx)

  tab_u = _repack(input_emb.T)
  tab_i = _repack(item_emb.T)

  pos_sc, neg_sc = _sc_scores(vrow_u, par_u, vrow_i, par_i, vrow_n, par_n,
                              tab_u, tab_i)

  out = pl.pallas_call(
      _tc_finish,
      out_shape=jax.ShapeDtypeStruct((1, 1), jnp.float32),
      in_specs=[
          pl.BlockSpec(memory_space=pltpu.VMEM),
          pl.BlockSpec(memory_space=pltpu.VMEM),
      ],
      out_specs=pl.BlockSpec(memory_space=pltpu.SMEM),
  )(pos_sc.reshape(128, 128), neg_sc.reshape(B * NNEG // 128, 128))
  return out[0, 0]
